# baseline (speedup 1.0000x reference)
import functools

import jax
import jax.numpy as jnp
from jax import lax
from jax.experimental import pallas as pl
from jax.experimental.pallas import tpu as pltpu
from jax.experimental.pallas import tpu_sc as plsc

_DD = 2.5
_GAMMA = 0.005
_K = 16
_HBLK = 48
_R0 = 336
_CROWS = 48
_NW = 32
_PLANES_PER_W = 12


def _sc_stats_body(feat_hbm, lab_hbm, sums_out, s2cnt_out,
                   lab_v, fbuf, acc_s, acc_qc, fold_s, fold_qc,
                   sem0, sem1):
    wid = lax.axis_index("s") * 2 + lax.axis_index("c")
    b = wid // 8
    c0 = (wid % 8) * _PLANES_PER_W

    zeros16 = jnp.zeros((16,), jnp.float32)
    ones16 = jnp.ones((16,), jnp.float32)
    lane = lax.iota(jnp.int32, 16)

    def _zero(i, _):
        acc_s[pl.ds(i * 16, 16)] = zeros16
        return 0
    lax.fori_loop(0, _PLANES_PER_W * 16, _zero, 0)

    def _zero2(i, _):
        acc_qc[pl.ds(i * 16, 16)] = zeros16
        return 0
    lax.fori_loop(0, 32, _zero2, 0)

    sems = (sem0, sem1)
    nchunks = (384 - _R0) // _CROWS

    def _chunk(chunk, _):
        row0 = _R0 + chunk * _CROWS
        pltpu.sync_copy(lab_hbm.at[b, pl.ds(row0, _CROWS), :], lab_v)
        copies = [None, None]
        copies[0] = pltpu.async_copy(
            feat_hbm.at[b, c0, pl.ds(row0, _CROWS), :], fbuf.at[0], sems[0])
        for j in range(_PLANES_PER_W):
            par = j % 2
            if j + 1 < _PLANES_PER_W:
                copies[1 - par] = pltpu.async_copy(
                    feat_hbm.at[b, c0 + j + 1, pl.ds(row0, _CROWS), :],
                    fbuf.at[1 - par], sems[1 - par])
            copies[par].wait()
            jbase = j * 256

            def _rows(r, _):
                @plsc.parallel_loop(0, 384, 16, unroll=8)
                def _col(col):
                    lab16 = lab_v[r, pl.ds(col, 16)]
                    f = fbuf[par, r, pl.ds(col, 16)]
                    labx = lab16 * 16 + lane
                    plsc.addupdate_scatter(acc_s, [jbase + labx], f)
                    plsc.addupdate_scatter(acc_qc, [labx], f * f)
                return 0
            lax.fori_loop(0, _CROWS, _rows, 0)

        @pl.when(c0 == 0)
        def _():
            def _crows(r, _):
                @plsc.parallel_loop(0, 384, 16, unroll=8)
                def _col(col):
                    lab16 = lab_v[r, pl.ds(col, 16)]
                    plsc.addupdate_scatter(
                        acc_qc, [256 + lab16 * 16 + lane], ones16)
                return 0
            lax.fori_loop(0, _CROWS, _crows, 0)
        return 0

    lax.fori_loop(0, nchunks, _chunk, 0)

    lane16 = lane * 16
    for j in range(_PLANES_PER_W):
        v = plsc.load_gather(acc_s, [j * 256 + lane16])
        for l in range(1, 16):
            v = v + plsc.load_gather(acc_s, [j * 256 + lane16 + l])
        fold_s[pl.ds(j * 16, 16)] = v
    for r in range(2):
        v = plsc.load_gather(acc_qc, [r * 256 + lane16])
        for l in range(1, 16):
            v = v + plsc.load_gather(acc_qc, [r * 256 + lane16 + l])
        fold_qc[pl.ds(r * 16, 16)] = v

    pltpu.sync_copy(fold_s, sums_out.at[wid])
    pltpu.sync_copy(fold_qc, s2cnt_out.at[wid])


def _sc_stats(features, labels):
    mesh = plsc.VectorSubcoreMesh(core_axis_name="c", subcore_axis_name="s")
    kern = functools.partial(
        pl.kernel,
        mesh=mesh,
        compiler_params=pltpu.CompilerParams(
            needs_layout_passes=False, use_tc_tiling_on_sc=True),
        out_type=[
            jax.ShapeDtypeStruct((_NW, _PLANES_PER_W * 16), jnp.float32),
            jax.ShapeDtypeStruct((_NW, 32), jnp.float32),
        ],
        scratch_types=[
            pltpu.VMEM((_CROWS, 384), jnp.int32),
            pltpu.VMEM((2, _CROWS, 384), jnp.float32),
            pltpu.VMEM((_PLANES_PER_W * 256,), jnp.float32),
            pltpu.VMEM((512,), jnp.float32),
            pltpu.VMEM((_PLANES_PER_W * 16,), jnp.float32),
            pltpu.VMEM((32,), jnp.float32),
            pltpu.SemaphoreType.DMA,
            pltpu.SemaphoreType.DMA,
        ],
    )(_sc_stats_body)
    return kern(features, labels)


def _stats_kernel(feat_ref, lab_ref, out_ref):
    h = pl.program_id(1)
    P = _HBLK * 384
    F = feat_ref[0]
    lab = lab_ref[0]

    classes = jax.lax.broadcasted_iota(jnp.int32, (_HBLK, 384, _K), 2)
    O = (lab[:, :, None] == classes).astype(jnp.bfloat16)
    O2 = O.reshape(P, _K)

    f2 = jnp.sum(F * F, axis=0, keepdims=True)
    ones = jnp.ones((1, _HBLK, 384), dtype=jnp.bfloat16)
    G = jnp.concatenate([f2.astype(jnp.bfloat16), ones], axis=0)
    G2 = G.reshape(2, P)
    Fb = F.astype(jnp.bfloat16).reshape(96, P)

    sums = jax.lax.dot_general(
        Fb, O2, (((1,), (0,)), ((), ())),
        preferred_element_type=jnp.float32)
    small = jax.lax.dot_general(
        G2, O2, (((1,), (0,)), ((), ())),
        preferred_element_type=jnp.float32)

    @pl.when(h == 0)
    def _():
        out_ref[...] = jnp.zeros_like(out_ref)

    out_ref[0, 0:96, 0:_K] += sums
    out_ref[0, 96:98, 0:_K] += small


def _loss_kernel(stats_ref, scs_ref, scq_ref, cntcol_ref, out_ref):
    total = 0.0
    for b in range(4):
        st = stats_ref[b]
        scq = scq_ref[b]
        sums = st[0:96, 0:_K] + scs_ref[b]
        s2 = st[96:97, 0:_K] + jnp.sum(scq[:, 0:1, :], axis=0)
        cnt = st[97:98, 0:_K] + jnp.sum(scq[:, 1:2, :], axis=0)
        cnt_col = cntcol_ref[b]

        present = cnt > 0.0
        cnt_safe = jnp.maximum(cnt, 1.0)
        means = sums / cnt_safe
        m2 = jnp.sum(means * means, axis=0, keepdims=True)
        var_per = (s2 - cnt * m2) / cnt_safe
        var_loss = jnp.sum(jnp.where(present, var_per, 0.0))
        num_clusters = jnp.sum(present.astype(jnp.float32))

        diff = means[:, :, None] - means[:, None, :]
        d2 = jnp.sum(diff * diff, axis=0)
        ii = jax.lax.broadcasted_iota(jnp.int32, (_K, _K), 0)
        jj = jax.lax.broadcasted_iota(jnp.int32, (_K, _K), 1)
        pres_row = jnp.broadcast_to(cnt_col > 0.0, (_K, _K))
        pres_col = jnp.broadcast_to(present, (_K, _K))
        pair_mask = (ii < jj) & pres_row & pres_col
        dist = jnp.sqrt(jnp.where(pair_mask, d2, 1.0))
        denom = jnp.maximum(num_clusters - 1.0, 1.0)
        pen = jnp.where(pair_mask & (dist < 2.0 * _DD),
                        (2.0 * _DD - dist) ** 2 / denom, 0.0)
        dist_loss = jnp.where(num_clusters > 1.0, jnp.sum(pen), 0.0)

        mnorm = jnp.sqrt(jnp.where(present, m2, 1.0))
        reg_loss = jnp.sum(jnp.where(present, mnorm, 0.0))

        total = total + (var_loss + dist_loss + _GAMMA * reg_loss) / num_clusters

    out_ref[...] = jnp.broadcast_to(total / 5.0, (1, 1))


def kernel(features_batch, labels_batch):
    B, C, H, W = features_batch.shape
    sc_sums_raw, sc_s2cnt = _sc_stats(features_batch, labels_batch)
    sc_sums = sc_sums_raw.reshape(B, C, 16)

    if _R0 > 0:
        stats = pl.pallas_call(
            _stats_kernel,
            grid=(B, _R0 // _HBLK),
            in_specs=[
                pl.BlockSpec((1, C, _HBLK, W), lambda b, h: (b, 0, h, 0)),
                pl.BlockSpec((1, _HBLK, W), lambda b, h: (b, h, 0)),
            ],
            out_specs=pl.BlockSpec((1, 104, 128), lambda b, h: (b, 0, 0)),
            out_shape=jax.ShapeDtypeStruct((B, 104, 128), jnp.float32),
        )(features_batch, labels_batch)
    else:
        stats = jnp.zeros((B, 104, 128), jnp.float32)

    scq = sc_s2cnt.reshape(B, 8, 2, 16)
    cnt_row = stats[:, 97, 0:_K] + scq[:, :, 1, :].sum(axis=1)
    cntcol = cnt_row.reshape(B, _K, 1)

    loss = pl.pallas_call(
        _loss_kernel,
        out_shape=jax.ShapeDtypeStruct((1, 1), jnp.float32),
    )(stats, sc_sums, scq, cntcol)
    return loss[0, 0]


# speedup vs baseline: 1.0998x; 1.0998x over previous
import functools

import jax
import jax.numpy as jnp
from jax import lax
from jax.experimental import pallas as pl
from jax.experimental.pallas import tpu as pltpu
from jax.experimental.pallas import tpu_sc as plsc

_DD = 2.5
_GAMMA = 0.005
_K = 16
_HBLK = 48
_R0 = 288
_CROWS = 48
_NW = 32
_PLANES_PER_W = 12


def _sc_stats_body(feat_hbm, lab_hbm, sums_out, s2cnt_out,
                   lab_v, fbuf, acc_s, acc_qc, fold_s, fold_qc,
                   sem0, sem1, sem2, sem3):
    wid = lax.axis_index("s") * 2 + lax.axis_index("c")
    b = wid // 8
    c0 = (wid % 8) * _PLANES_PER_W

    zeros16 = jnp.zeros((16,), jnp.float32)
    ones16 = jnp.ones((16,), jnp.float32)
    lane = lax.iota(jnp.int32, 16)

    def _zero(i, _):
        acc_s[pl.ds(i * 16, 16)] = zeros16
        return 0
    lax.fori_loop(0, _PLANES_PER_W * 16, _zero, 0)

    def _zero2(i, _):
        acc_qc[pl.ds(i * 16, 16)] = zeros16
        return 0
    lax.fori_loop(0, 32, _zero2, 0)

    sems = (sem0, sem1, sem2, sem3)
    nchunks = (384 - _R0) // _CROWS
    _NB = 4

    def _chunk(chunk, _):
        row0 = _R0 + chunk * _CROWS
        pltpu.sync_copy(lab_hbm.at[b, pl.ds(row0, _CROWS), :], lab_v)
        copies = [None] * _NB
        for j0 in range(_NB - 1):
            copies[j0] = pltpu.async_copy(
                feat_hbm.at[b, c0 + j0, pl.ds(row0, _CROWS), :],
                fbuf.at[j0], sems[j0])
        for j in range(_PLANES_PER_W):
            par = j % _NB
            if j + _NB - 1 < _PLANES_PER_W:
                nxt = (j + _NB - 1) % _NB
                copies[nxt] = pltpu.async_copy(
                    feat_hbm.at[b, c0 + j + _NB - 1, pl.ds(row0, _CROWS), :],
                    fbuf.at[nxt], sems[nxt])
            copies[par].wait()
            jbase = j * 256

            def _rows(r, _):
                @plsc.parallel_loop(0, 384, 16, unroll=8)
                def _col(col):
                    lab16 = lab_v[r, pl.ds(col, 16)]
                    f = fbuf[par, r, pl.ds(col, 16)]
                    labx = lab16 * 16 + lane
                    plsc.addupdate_scatter(acc_s, [jbase + labx], f)
                    plsc.addupdate_scatter(acc_qc, [labx], f * f)
                return 0
            lax.fori_loop(0, _CROWS, _rows, 0)

        @pl.when(c0 == 0)
        def _():
            def _crows(r, _):
                @plsc.parallel_loop(0, 384, 16, unroll=8)
                def _col(col):
                    lab16 = lab_v[r, pl.ds(col, 16)]
                    plsc.addupdate_scatter(
                        acc_qc, [256 + lab16 * 16 + lane], ones16)
                return 0
            lax.fori_loop(0, _CROWS, _crows, 0)
        return 0

    lax.fori_loop(0, nchunks, _chunk, 0)

    lane16 = lane * 16
    for j in range(_PLANES_PER_W):
        v = plsc.load_gather(acc_s, [j * 256 + lane16])
        for l in range(1, 16):
            v = v + plsc.load_gather(acc_s, [j * 256 + lane16 + l])
        fold_s[pl.ds(j * 16, 16)] = v
    for r in range(2):
        v = plsc.load_gather(acc_qc, [r * 256 + lane16])
        for l in range(1, 16):
            v = v + plsc.load_gather(acc_qc, [r * 256 + lane16 + l])
        fold_qc[pl.ds(r * 16, 16)] = v

    pltpu.sync_copy(fold_s, sums_out.at[wid])
    pltpu.sync_copy(fold_qc, s2cnt_out.at[wid])


def _sc_stats(features, labels):
    mesh = plsc.VectorSubcoreMesh(core_axis_name="c", subcore_axis_name="s")
    kern = functools.partial(
        pl.kernel,
        mesh=mesh,
        compiler_params=pltpu.CompilerParams(
            needs_layout_passes=False, use_tc_tiling_on_sc=True),
        out_type=[
            jax.ShapeDtypeStruct((_NW, _PLANES_PER_W * 16), jnp.float32),
            jax.ShapeDtypeStruct((_NW, 32), jnp.float32),
        ],
        scratch_types=[
            pltpu.VMEM((_CROWS, 384), jnp.int32),
            pltpu.VMEM((4, _CROWS, 384), jnp.float32),
            pltpu.VMEM((_PLANES_PER_W * 256,), jnp.float32),
            pltpu.VMEM((512,), jnp.float32),
            pltpu.VMEM((_PLANES_PER_W * 16,), jnp.float32),
            pltpu.VMEM((32,), jnp.float32),
            pltpu.SemaphoreType.DMA,
            pltpu.SemaphoreType.DMA,
            pltpu.SemaphoreType.DMA,
            pltpu.SemaphoreType.DMA,
        ],
    )(_sc_stats_body)
    return kern(features, labels)


def _stats_kernel(feat_ref, lab_ref, out_ref):
    h = pl.program_id(1)
    P = _HBLK * 384
    F = feat_ref[0]
    lab = lab_ref[0]

    classes = jax.lax.broadcasted_iota(jnp.int32, (_HBLK, 384, _K), 2)
    O = (lab[:, :, None] == classes).astype(jnp.bfloat16)
    O2 = O.reshape(P, _K)

    f2 = jnp.sum(F * F, axis=0, keepdims=True)
    ones = jnp.ones((1, _HBLK, 384), dtype=jnp.bfloat16)
    G = jnp.concatenate([f2.astype(jnp.bfloat16), ones], axis=0)
    G2 = G.reshape(2, P)
    Fb = F.astype(jnp.bfloat16).reshape(96, P)

    sums = jax.lax.dot_general(
        Fb, O2, (((1,), (0,)), ((), ())),
        preferred_element_type=jnp.float32)
    small = jax.lax.dot_general(
        G2, O2, (((1,), (0,)), ((), ())),
        preferred_element_type=jnp.float32)

    @pl.when(h == 0)
    def _():
        out_ref[...] = jnp.zeros_like(out_ref)

    out_ref[0, 0:96, 0:_K] += sums
    out_ref[0, 96:98, 0:_K] += small


def _loss_kernel(stats_ref, scs_ref, scq_ref, cntcol_ref, out_ref):
    total = 0.0
    for b in range(4):
        st = stats_ref[b]
        scq = scq_ref[b]
        sums = st[0:96, 0:_K] + scs_ref[b]
        s2 = st[96:97, 0:_K] + jnp.sum(scq[:, 0:1, :], axis=0)
        cnt = st[97:98, 0:_K] + jnp.sum(scq[:, 1:2, :], axis=0)
        cnt_col = cntcol_ref[b]

        present = cnt > 0.0
        cnt_safe = jnp.maximum(cnt, 1.0)
        means = sums / cnt_safe
        m2 = jnp.sum(means * means, axis=0, keepdims=True)
        var_per = (s2 - cnt * m2) / cnt_safe
        var_loss = jnp.sum(jnp.where(present, var_per, 0.0))
        num_clusters = jnp.sum(present.astype(jnp.float32))

        diff = means[:, :, None] - means[:, None, :]
        d2 = jnp.sum(diff * diff, axis=0)
        ii = jax.lax.broadcasted_iota(jnp.int32, (_K, _K), 0)
        jj = jax.lax.broadcasted_iota(jnp.int32, (_K, _K), 1)
        pres_row = jnp.broadcast_to(cnt_col > 0.0, (_K, _K))
        pres_col = jnp.broadcast_to(present, (_K, _K))
        pair_mask = (ii < jj) & pres_row & pres_col
        dist = jnp.sqrt(jnp.where(pair_mask, d2, 1.0))
        denom = jnp.maximum(num_clusters - 1.0, 1.0)
        pen = jnp.where(pair_mask & (dist < 2.0 * _DD),
                        (2.0 * _DD - dist) ** 2 / denom, 0.0)
        dist_loss = jnp.where(num_clusters > 1.0, jnp.sum(pen), 0.0)

        mnorm = jnp.sqrt(jnp.where(present, m2, 1.0))
        reg_loss = jnp.sum(jnp.where(present, mnorm, 0.0))

        total = total + (var_loss + dist_loss + _GAMMA * reg_loss) / num_clusters

    out_ref[...] = jnp.broadcast_to(total / 5.0, (1, 1))


def kernel(features_batch, labels_batch):
    B, C, H, W = features_batch.shape
    sc_sums_raw, sc_s2cnt = _sc_stats(features_batch, labels_batch)
    sc_sums = sc_sums_raw.reshape(B, C, 16)

    if _R0 > 0:
        stats = pl.pallas_call(
            _stats_kernel,
            grid=(B, _R0 // _HBLK),
            in_specs=[
                pl.BlockSpec((1, C, _HBLK, W), lambda b, h: (b, 0, h, 0)),
                pl.BlockSpec((1, _HBLK, W), lambda b, h: (b, h, 0)),
            ],
            out_specs=pl.BlockSpec((1, 104, 128), lambda b, h: (b, 0, 0)),
            out_shape=jax.ShapeDtypeStruct((B, 104, 128), jnp.float32),
        )(features_batch, labels_batch)
    else:
        stats = jnp.zeros((B, 104, 128), jnp.float32)

    scq = sc_s2cnt.reshape(B, 8, 2, 16)
    cnt_row = stats[:, 97, 0:_K] + scq[:, :, 1, :].sum(axis=1)
    cntcol = cnt_row.reshape(B, _K, 1)

    loss = pl.pallas_call(
        _loss_kernel,
        out_shape=jax.ShapeDtypeStruct((1, 1), jnp.float32),
    )(stats, sc_sums, scq, cntcol)
    return loss[0, 0]


# speedup vs baseline: 1.1046x; 1.0044x over previous
import functools

import jax
import jax.numpy as jnp
from jax import lax
from jax.experimental import pallas as pl
from jax.experimental.pallas import tpu as pltpu
from jax.experimental.pallas import tpu_sc as plsc

_DD = 2.5
_GAMMA = 0.005
_K = 16
_HBLK = 48
_R0 = 288
_CROWS = 48
_NW = 32
_PLANES_PER_W = 12


def _sc_stats_body(feat_hbm, lab_hbm, sums_out, s2cnt_out,
                   lab_v, fbuf, acc_s, acc_qc, fold_s, fold_qc,
                   sem0, sem1, sem2, sem3):
    wid = lax.axis_index("s") * 2 + lax.axis_index("c")
    b = wid // 8
    c0 = (wid % 8) * _PLANES_PER_W

    zeros16 = jnp.zeros((16,), jnp.float32)
    ones16 = jnp.ones((16,), jnp.float32)
    lane = lax.iota(jnp.int32, 16)

    def _zero(i, _):
        acc_s[pl.ds(i * 16, 16)] = zeros16
        return 0
    lax.fori_loop(0, _PLANES_PER_W * 16, _zero, 0)

    def _zero2(i, _):
        acc_qc[pl.ds(i * 16, 16)] = zeros16
        return 0
    lax.fori_loop(0, 32, _zero2, 0)

    sems = (sem0, sem1, sem2, sem3)
    nchunks = (384 - _R0) // _CROWS
    _NB = 4

    def _chunk(chunk, _):
        row0 = _R0 + chunk * _CROWS
        pltpu.sync_copy(lab_hbm.at[b, pl.ds(row0, _CROWS), :], lab_v)
        copies = [None] * _NB
        for j0 in range(_NB - 1):
            copies[j0] = pltpu.async_copy(
                feat_hbm.at[b, c0 + j0, pl.ds(row0, _CROWS), :],
                fbuf.at[j0], sems[j0])
        for j in range(_PLANES_PER_W):
            par = j % _NB
            if j + _NB - 1 < _PLANES_PER_W:
                nxt = (j + _NB - 1) % _NB
                copies[nxt] = pltpu.async_copy(
                    feat_hbm.at[b, c0 + j + _NB - 1, pl.ds(row0, _CROWS), :],
                    fbuf.at[nxt], sems[nxt])
            copies[par].wait()
            jbase = j * 256

            def _rows(r, _):
                @plsc.parallel_loop(0, 384, 16, unroll=8)
                def _col(col):
                    lab16 = lab_v[r, pl.ds(col, 16)]
                    f = fbuf[par, r, pl.ds(col, 16)]
                    labx = lab16 * 16 + lane
                    plsc.addupdate_scatter(acc_s, [jbase + labx], f)
                    plsc.addupdate_scatter(acc_qc, [labx], f * f)
                return 0
            lax.fori_loop(0, _CROWS, _rows, 0)

        @pl.when(c0 == 0)
        def _():
            def _crows(r, _):
                @plsc.parallel_loop(0, 384, 16, unroll=8)
                def _col(col):
                    lab16 = lab_v[r, pl.ds(col, 16)]
                    plsc.addupdate_scatter(
                        acc_qc, [256 + lab16 * 16 + lane], ones16)
                return 0
            lax.fori_loop(0, _CROWS, _crows, 0)
        return 0

    lax.fori_loop(0, nchunks, _chunk, 0)

    lane16 = lane * 16
    for j in range(_PLANES_PER_W):
        v = plsc.load_gather(acc_s, [j * 256 + lane16])
        for l in range(1, 16):
            v = v + plsc.load_gather(acc_s, [j * 256 + lane16 + l])
        fold_s[pl.ds(j * 16, 16)] = v
    for r in range(2):
        v = plsc.load_gather(acc_qc, [r * 256 + lane16])
        for l in range(1, 16):
            v = v + plsc.load_gather(acc_qc, [r * 256 + lane16 + l])
        fold_qc[pl.ds(r * 16, 16)] = v

    pltpu.sync_copy(fold_s, sums_out.at[wid])
    pltpu.sync_copy(fold_qc, s2cnt_out.at[wid])


def _sc_stats(features, labels):
    mesh = plsc.VectorSubcoreMesh(core_axis_name="c", subcore_axis_name="s")
    kern = functools.partial(
        pl.kernel,
        mesh=mesh,
        compiler_params=pltpu.CompilerParams(
            needs_layout_passes=False, use_tc_tiling_on_sc=True),
        out_type=[
            jax.ShapeDtypeStruct((_NW, _PLANES_PER_W * 16), jnp.float32),
            jax.ShapeDtypeStruct((_NW, 32), jnp.float32),
        ],
        scratch_types=[
            pltpu.VMEM((_CROWS, 384), jnp.int32),
            pltpu.VMEM((4, _CROWS, 384), jnp.float32),
            pltpu.VMEM((_PLANES_PER_W * 256,), jnp.float32),
            pltpu.VMEM((512,), jnp.float32),
            pltpu.VMEM((_PLANES_PER_W * 16,), jnp.float32),
            pltpu.VMEM((32,), jnp.float32),
            pltpu.SemaphoreType.DMA,
            pltpu.SemaphoreType.DMA,
            pltpu.SemaphoreType.DMA,
            pltpu.SemaphoreType.DMA,
        ],
    )(_sc_stats_body)
    return kern(features, labels)


def _stats_kernel(feat_ref, lab_ref, out_ref):
    h = pl.program_id(1)
    P = _HBLK * 384
    F = feat_ref[0]
    lab = lab_ref[0]

    classes = jax.lax.broadcasted_iota(jnp.int32, (_HBLK, 384, _K), 2)
    O = (lab[:, :, None] == classes).astype(jnp.float32)
    O2 = O.reshape(P, _K)

    f2 = jnp.sum(F * F, axis=0, keepdims=True)
    ones = jnp.ones((1, _HBLK, 384), dtype=jnp.float32)
    G = jnp.concatenate([F, f2, ones], axis=0)
    G2 = G.reshape(98, P)

    stats = jax.lax.dot_general(
        G2, O2, (((1,), (0,)), ((), ())),
        preferred_element_type=jnp.float32)

    @pl.when(h == 0)
    def _():
        out_ref[...] = jnp.zeros_like(out_ref)

    out_ref[0, 0:98, 0:_K] += stats


def _loss_kernel(stats_ref, scs_ref, scq_ref, out_ref):
    iia = jax.lax.broadcasted_iota(jnp.int32, (_K, _K), 0)
    jja = jax.lax.broadcasted_iota(jnp.int32, (_K, _K), 1)
    eye = (iia == jja).astype(jnp.float32)
    total = 0.0
    for b in range(4):
        st = stats_ref[b]
        scq = scq_ref[b]
        sums = st[0:96, 0:_K] + scs_ref[b]
        s2 = st[96:97, 0:_K] + jnp.sum(scq[:, 0:1, :], axis=0)
        cnt = st[97:98, 0:_K] + jnp.sum(scq[:, 1:2, :], axis=0)
        cnt_mat = jnp.broadcast_to(cnt, (_K, _K))
        cnt_t = jax.lax.dot_general(
            eye, cnt_mat, (((1,), (1,)), ((), ())),
            preferred_element_type=jnp.float32)

        present = cnt > 0.0
        cnt_safe = jnp.maximum(cnt, 1.0)
        means = sums / cnt_safe
        m2 = jnp.sum(means * means, axis=0, keepdims=True)
        var_per = (s2 - cnt * m2) / cnt_safe
        var_loss = jnp.sum(jnp.where(present, var_per, 0.0))
        num_clusters = jnp.sum(present.astype(jnp.float32))

        diff = means[:, :, None] - means[:, None, :]
        d2 = jnp.sum(diff * diff, axis=0)
        ii = jax.lax.broadcasted_iota(jnp.int32, (_K, _K), 0)
        jj = jax.lax.broadcasted_iota(jnp.int32, (_K, _K), 1)
        pres_row = cnt_t > 0.0
        pres_col = jnp.broadcast_to(present, (_K, _K))
        pair_mask = (ii < jj) & pres_row & pres_col
        dist = jnp.sqrt(jnp.where(pair_mask, d2, 1.0))
        denom = jnp.maximum(num_clusters - 1.0, 1.0)
        pen = jnp.where(pair_mask & (dist < 2.0 * _DD),
                        (2.0 * _DD - dist) ** 2 / denom, 0.0)
        dist_loss = jnp.where(num_clusters > 1.0, jnp.sum(pen), 0.0)

        mnorm = jnp.sqrt(jnp.where(present, m2, 1.0))
        reg_loss = jnp.sum(jnp.where(present, mnorm, 0.0))

        total = total + (var_loss + dist_loss + _GAMMA * reg_loss) / num_clusters

    out_ref[...] = jnp.broadcast_to(total / 5.0, (1, 1))


def kernel(features_batch, labels_batch):
    B, C, H, W = features_batch.shape
    sc_sums_raw, sc_s2cnt = _sc_stats(features_batch, labels_batch)
    sc_sums = sc_sums_raw.reshape(B, C, 16)

    if _R0 > 0:
        stats = pl.pallas_call(
            _stats_kernel,
            grid=(B, _R0 // _HBLK),
            in_specs=[
                pl.BlockSpec((1, C, _HBLK, W), lambda b, h: (b, 0, h, 0)),
                pl.BlockSpec((1, _HBLK, W), lambda b, h: (b, h, 0)),
            ],
            out_specs=pl.BlockSpec((1, 104, 128), lambda b, h: (b, 0, 0)),
            out_shape=jax.ShapeDtypeStruct((B, 104, 128), jnp.float32),
        )(features_batch, labels_batch)
    else:
        stats = jnp.zeros((B, 104, 128), jnp.float32)

    scq = sc_s2cnt.reshape(B, 8, 2, 16)

    loss = pl.pallas_call(
        _loss_kernel,
        out_shape=jax.ShapeDtypeStruct((1, 1), jnp.float32),
    )(stats, sc_sums, scq)
    return loss[0, 0]


# speedup vs baseline: 1.1557x; 1.0463x over previous
import functools

import jax
import jax.numpy as jnp
from jax import lax
from jax.experimental import pallas as pl
from jax.experimental.pallas import tpu as pltpu
from jax.experimental.pallas import tpu_sc as plsc

_DD = 2.5
_GAMMA = 0.005
_K = 16
_HBLK = 48
_R0 = 288
_CROWS = 48
_NW = 32
_PLANES_PER_W = 12


def _sc_stats_body(feat_hbm, lab_hbm, sums_out, s2cnt_out,
                   lab_v, fbuf, acc_s, acc_qc, fold_s, fold_qc,
                   sem0, sem1, sem2, sem3):
    wid = lax.axis_index("s") * 2 + lax.axis_index("c")
    b = wid // 8
    c0 = (wid % 8) * _PLANES_PER_W

    zeros16 = jnp.zeros((16,), jnp.float32)
    ones16 = jnp.ones((16,), jnp.float32)
    lane = lax.iota(jnp.int32, 16)

    def _zero(i, _):
        acc_s[pl.ds(i * 16, 16)] = zeros16
        return 0
    lax.fori_loop(0, _PLANES_PER_W * 16, _zero, 0)

    def _zero2(i, _):
        acc_qc[pl.ds(i * 16, 16)] = zeros16
        return 0
    lax.fori_loop(0, 32, _zero2, 0)

    sems = (sem0, sem1, sem2, sem3)
    nchunks = (384 - _R0) // _CROWS
    _NB = 2

    def _chunk(chunk, _):
        row0 = _R0 + chunk * _CROWS
        pltpu.sync_copy(lab_hbm.at[b, pl.ds(row0, _CROWS), :], lab_v)
        copies = [None] * _NB
        for j0 in range(_NB - 1):
            copies[j0] = pltpu.async_copy(
                feat_hbm.at[b, c0 + j0, pl.ds(row0, _CROWS), :],
                fbuf.at[j0], sems[j0])
        for j in range(_PLANES_PER_W):
            par = j % _NB
            if j + _NB - 1 < _PLANES_PER_W:
                nxt = (j + _NB - 1) % _NB
                copies[nxt] = pltpu.async_copy(
                    feat_hbm.at[b, c0 + j + _NB - 1, pl.ds(row0, _CROWS), :],
                    fbuf.at[nxt], sems[nxt])
            copies[par].wait()
            jbase = j * 256

            def _rows(r, _):
                @plsc.parallel_loop(0, 384, 16, unroll=8)
                def _col(col):
                    lab16 = lab_v[r, pl.ds(col, 16)]
                    f = fbuf[par, r, pl.ds(col, 16)]
                    labx = lab16 * 16 + lane
                    plsc.addupdate_scatter(acc_s, [jbase + labx], f)
                    plsc.addupdate_scatter(acc_qc, [labx], f * f)
                return 0
            lax.fori_loop(0, _CROWS, _rows, 0)

        @pl.when(c0 == 0)
        def _():
            def _crows(r, _):
                @plsc.parallel_loop(0, 384, 16, unroll=8)
                def _col(col):
                    lab16 = lab_v[r, pl.ds(col, 16)]
                    plsc.addupdate_scatter(
                        acc_qc, [256 + lab16 * 16 + lane], ones16)
                return 0
            lax.fori_loop(0, _CROWS, _crows, 0)
        return 0

    lax.fori_loop(0, nchunks, _chunk, 0)

    lane16 = lane * 16
    for j in range(_PLANES_PER_W):
        v = plsc.load_gather(acc_s, [j * 256 + lane16])
        for l in range(1, 16):
            v = v + plsc.load_gather(acc_s, [j * 256 + lane16 + l])
        fold_s[pl.ds(j * 16, 16)] = v
    for r in range(2):
        v = plsc.load_gather(acc_qc, [r * 256 + lane16])
        for l in range(1, 16):
            v = v + plsc.load_gather(acc_qc, [r * 256 + lane16 + l])
        fold_qc[pl.ds(r * 16, 16)] = v

    pltpu.sync_copy(fold_s, sums_out.at[wid])
    pltpu.sync_copy(fold_qc, s2cnt_out.at[wid])


def _sc_stats(features, labels):
    mesh = plsc.VectorSubcoreMesh(core_axis_name="c", subcore_axis_name="s")
    kern = functools.partial(
        pl.kernel,
        mesh=mesh,
        compiler_params=pltpu.CompilerParams(
            needs_layout_passes=False, use_tc_tiling_on_sc=True),
        out_type=[
            jax.ShapeDtypeStruct((_NW, _PLANES_PER_W * 16), jnp.float32),
            jax.ShapeDtypeStruct((_NW, 32), jnp.float32),
        ],
        scratch_types=[
            pltpu.VMEM((_CROWS, 384), jnp.int32),
            pltpu.VMEM((4, _CROWS, 384), jnp.float32),
            pltpu.VMEM((_PLANES_PER_W * 256,), jnp.float32),
            pltpu.VMEM((512,), jnp.float32),
            pltpu.VMEM((_PLANES_PER_W * 16,), jnp.float32),
            pltpu.VMEM((32,), jnp.float32),
            pltpu.SemaphoreType.DMA,
            pltpu.SemaphoreType.DMA,
            pltpu.SemaphoreType.DMA,
            pltpu.SemaphoreType.DMA,
        ],
    )(_sc_stats_body)
    return kern(features, labels)


def _stats_kernel(feat_ref, lab_ref, out_ref):
    h = pl.program_id(1)
    P = _HBLK * 384
    F = feat_ref[0]
    lab = lab_ref[0]

    classes = jax.lax.broadcasted_iota(jnp.int32, (_HBLK, 384, _K), 2)
    O = (lab[:, :, None] == classes).astype(jnp.float32)
    O2 = O.reshape(P, _K)

    f2 = jnp.sum(F * F, axis=0, keepdims=True)
    ones = jnp.ones((1, _HBLK, 384), dtype=jnp.float32)
    G = jnp.concatenate([F, f2, ones], axis=0)
    G2 = G.reshape(98, P)

    stats = jax.lax.dot_general(
        G2, O2, (((1,), (0,)), ((), ())),
        preferred_element_type=jnp.float32)

    @pl.when(h == 0)
    def _():
        out_ref[...] = jnp.zeros_like(out_ref)

    out_ref[0, 0:98, 0:_K] += stats


def _loss_kernel(stats_ref, scs_ref, scq_ref, out_ref):
    iia = jax.lax.broadcasted_iota(jnp.int32, (_K, _K), 0)
    jja = jax.lax.broadcasted_iota(jnp.int32, (_K, _K), 1)
    eye = (iia == jja).astype(jnp.float32)
    total = 0.0
    for b in range(4):
        st = stats_ref[b]
        scq = scq_ref[b]
        sums = st[0:96, 0:_K] + scs_ref[b]
        s2 = st[96:97, 0:_K] + jnp.sum(scq[:, 0:1, :], axis=0)
        cnt = st[97:98, 0:_K] + jnp.sum(scq[:, 1:2, :], axis=0)
        cnt_mat = jnp.broadcast_to(cnt, (_K, _K))
        cnt_t = jax.lax.dot_general(
            eye, cnt_mat, (((1,), (1,)), ((), ())),
            preferred_element_type=jnp.float32)

        present = cnt > 0.0
        cnt_safe = jnp.maximum(cnt, 1.0)
        means = sums / cnt_safe
        m2 = jnp.sum(means * means, axis=0, keepdims=True)
        var_per = (s2 - cnt * m2) / cnt_safe
        var_loss = jnp.sum(jnp.where(present, var_per, 0.0))
        num_clusters = jnp.sum(present.astype(jnp.float32))

        diff = means[:, :, None] - means[:, None, :]
        d2 = jnp.sum(diff * diff, axis=0)
        ii = jax.lax.broadcasted_iota(jnp.int32, (_K, _K), 0)
        jj = jax.lax.broadcasted_iota(jnp.int32, (_K, _K), 1)
        pres_row = cnt_t > 0.0
        pres_col = jnp.broadcast_to(present, (_K, _K))
        pair_mask = (ii < jj) & pres_row & pres_col
        dist = jnp.sqrt(jnp.where(pair_mask, d2, 1.0))
        denom = jnp.maximum(num_clusters - 1.0, 1.0)
        pen = jnp.where(pair_mask & (dist < 2.0 * _DD),
                        (2.0 * _DD - dist) ** 2 / denom, 0.0)
        dist_loss = jnp.where(num_clusters > 1.0, jnp.sum(pen), 0.0)

        mnorm = jnp.sqrt(jnp.where(present, m2, 1.0))
        reg_loss = jnp.sum(jnp.where(present, mnorm, 0.0))

        total = total + (var_loss + dist_loss + _GAMMA * reg_loss) / num_clusters

    out_ref[...] = jnp.broadcast_to(total / 5.0, (1, 1))


def kernel(features_batch, labels_batch):
    B, C, H, W = features_batch.shape
    sc_sums_raw, sc_s2cnt = _sc_stats(features_batch, labels_batch)
    sc_sums = sc_sums_raw.reshape(B, C, 16)

    if _R0 > 0:
        stats = pl.pallas_call(
            _stats_kernel,
            grid=(B, _R0 // _HBLK),
            in_specs=[
                pl.BlockSpec((1, C, _HBLK, W), lambda b, h: (b, 0, h, 0)),
                pl.BlockSpec((1, _HBLK, W), lambda b, h: (b, h, 0)),
            ],
            out_specs=pl.BlockSpec((1, 104, 128), lambda b, h: (b, 0, 0)),
            out_shape=jax.ShapeDtypeStruct((B, 104, 128), jnp.float32),
        )(features_batch, labels_batch)
    else:
        stats = jnp.zeros((B, 104, 128), jnp.float32)

    scq = sc_s2cnt.reshape(B, 8, 2, 16)

    loss = pl.pallas_call(
        _loss_kernel,
        out_shape=jax.ShapeDtypeStruct((1, 1), jnp.float32),
    )(stats, sc_sums, scq)
    return loss[0, 0]
